# baseline (speedup 1.0000x reference)
import functools

import jax
import jax.numpy as jnp
from jax import lax
from jax.experimental import pallas as pl
from jax.experimental.pallas import tpu as pltpu
from jax.experimental.pallas import tpu_sc as plsc

D = 64
SCALE = 8.0
NC = 2
NS = 16
L = 16
NW = NC * NS
BB = 128


def _make_kernel(T: int, NB: int, V: int):
  mesh = plsc.VectorSubcoreMesh(
      core_axis_name="c", subcore_axis_name="s",
      num_cores=NC, num_subcores=NS)

  NG = 4
  scratch = (
      [pltpu.VMEM((T, BB), jnp.int32)]
      + [pltpu.VMEM((BB,), jnp.int32) for _ in range(NG)]
      + [pltpu.VMEM((BB, 2 * D), jnp.float32) for _ in range(NG)]
      + [pltpu.VMEM((D, BB), jnp.float32) for _ in range(2)]
      + [pltpu.SemaphoreType.DMA for _ in range(NG + 2)]
  )

  @functools.partial(
      pl.kernel,
      mesh=mesh,
      out_type=jax.ShapeDtypeStruct((T, D, NB), jnp.float32),
      scratch_types=scratch,
      compiler_params=pltpu.CompilerParams(
          use_tc_tiling_on_sc=True, needs_layout_passes=False),
  )
  def emb(idxT_hbm, tab_hbm, out_hbm, itile, *rest):
    NGB = 4
    gidx = rest[0:NGB]
    gbufs = rest[NGB:2 * NGB]
    obufs = rest[2 * NGB:2 * NGB + 2]
    sem_g = rest[2 * NGB + 2:3 * NGB + 2]
    sem_o = rest[3 * NGB + 2:3 * NGB + 4]

    wid = lax.axis_index("s") * NC + lax.axis_index("c")
    bbase = wid * BB
    iota = lax.iota(jnp.int32, L)

    def fire_gather(s, gi, gb, sem):
      for lo in range(BB // L):
        iv = itile[s, pl.ds(L * lo, L)]
        gi[pl.ds(L * lo, L)] = lax.shift_right_logical(iv, 1)
      pltpu.async_copy(tab_hbm.at[gi], gb, sem)

    def extract(s, gbuf, obuf):
      for lo in range(BB // L):
        rows = iota + (L * lo)
        off = (itile[s, pl.ds(L * lo, L)] & 1) * D
        for j0 in range(0, D, L):

          def kb(k, rows=rows, off=off, j0=j0, gbuf=gbuf, obuf=obuf):
            jj = ((iota + k) & (L - 1)) + j0
            v = plsc.load_gather(gbuf, [rows, off + jj])
            plsc.store_scatter(obuf, [jj, rows], v * SCALE)

          plsc.parallel_loop(0, L, unroll=4)(kb)

    pltpu.sync_copy(idxT_hbm.at[:, pl.ds(bbase, BB)], itile)
    for b in range(NGB - 1):
      fire_gather(b, gidx[b], gbufs[b], sem_g[b])

    def group_body(g, carry):
      for b in range(NGB):
        s = NGB * g + b
        nb = (b + NGB - 1) % NGB
        ob = b & 1

        @pl.when(s + NGB - 1 < T)
        def _(s=s, nb=nb):
          fire_gather(s + NGB - 1, gidx[nb], gbufs[nb], sem_g[nb])

        pltpu.make_async_copy(
            tab_hbm.at[gidx[b]], gbufs[b], sem_g[b]).wait()

        @pl.when(s >= 2)
        def _(ob=ob):
          pltpu.make_async_copy(
              obufs[ob], out_hbm.at[0, :, pl.ds(bbase, BB)], sem_o[ob]).wait()

        extract(s, gbufs[b], obufs[ob])
        pltpu.async_copy(
            obufs[ob], out_hbm.at[s, :, pl.ds(bbase, BB)], sem_o[ob])
      return carry

    lax.fori_loop(0, T // NGB, group_body, 0)

    for ob in range(2):
      pltpu.make_async_copy(
          obufs[ob], out_hbm.at[0, :, pl.ds(bbase, BB)], sem_o[ob]).wait()

  return emb


def kernel(inputs, table):
  NB, T = inputs.shape
  V, d = table.shape
  assert d == D and V % 2 == 0 and NB == NW * BB and T % 2 == 0
  idxT = inputs.T
  tab2 = table.reshape(V // 2, 2 * D)
  out = _make_kernel(T, NB, V)(idxT, tab2)
  return jnp.transpose(out, (2, 0, 1))


# speedup vs baseline: 1.0785x; 1.0785x over previous
import functools

import jax
import jax.numpy as jnp
from jax import lax
from jax.experimental import pallas as pl
from jax.experimental.pallas import tpu as pltpu
from jax.experimental.pallas import tpu_sc as plsc

D = 64
SCALE = 8.0
NC = 2
NS = 16
L = 16
NW = NC * NS
BB = 128
NG = 2


def _make_kernel(T: int, NB: int):
  mesh = plsc.VectorSubcoreMesh(
      core_axis_name="c", subcore_axis_name="s",
      num_cores=NC, num_subcores=NS)

  scratch = (
      [pltpu.VMEM((T, BB), jnp.int32)]
      + [pltpu.VMEM((BB, 2 * D), jnp.float32) for _ in range(NG)]
      + [pltpu.VMEM((D, BB), jnp.float32) for _ in range(2)]
      + [pltpu.SemaphoreType.DMA for _ in range(NG + 2)]
  )

  @functools.partial(
      pl.kernel,
      mesh=mesh,
      out_type=jax.ShapeDtypeStruct((T, D, NB), jnp.float32),
      scratch_types=scratch,
      compiler_params=pltpu.CompilerParams(
          use_tc_tiling_on_sc=True, needs_layout_passes=False),
  )
  def emb(idxT_hbm, tab_hbm, out_hbm, itile, *rest):
    gbufs = rest[0:NG]
    obufs = rest[NG:NG + 2]
    sem_g = rest[NG + 2:2 * NG + 2]
    sem_o = rest[2 * NG + 2:2 * NG + 4]

    wid = lax.axis_index("s") * NC + lax.axis_index("c")
    bbase = wid * BB
    iota = lax.iota(jnp.int32, L)

    def fire_gather(s, gb, sem):
      pltpu.async_copy(tab_hbm.at[itile.at[s]], gb, sem)

    def extract(s, gbuf, obuf):
      for lo in range(BB // L):
        rows = iota + (L * lo)
        for j0 in range(0, D, L):

          def kb(k, rows=rows, j0=j0, gbuf=gbuf, obuf=obuf):
            jj = ((iota + k) & (L - 1)) + j0
            v = plsc.load_gather(gbuf, [rows, jj])
            plsc.store_scatter(obuf, [jj, rows], v * SCALE)

          plsc.parallel_loop(0, L, unroll=4)(kb)

    pltpu.sync_copy(idxT_hbm.at[:, pl.ds(bbase, BB)], itile)
    for b in range(NG - 1):
      fire_gather(b, gbufs[b], sem_g[b])

    def group_body(g, carry):
      for b in range(NG):
        s = NG * g + b
        nb = (b + NG - 1) % NG
        ob = b & 1

        @pl.when(s + NG - 1 < T)
        def _(s=s, nb=nb):
          fire_gather(s + NG - 1, gbufs[nb], sem_g[nb])

        pltpu.make_async_copy(
            tab_hbm.at[itile.at[0]], gbufs[b], sem_g[b]).wait()

        @pl.when(s >= 2)
        def _(ob=ob):
          pltpu.make_async_copy(
              obufs[ob], out_hbm.at[0, :, pl.ds(bbase, BB)], sem_o[ob]).wait()

        extract(s, gbufs[b], obufs[ob])
        pltpu.async_copy(
            obufs[ob], out_hbm.at[s, :, pl.ds(bbase, BB)], sem_o[ob])
      return carry

    lax.fori_loop(0, T // NG, group_body, 0)

    for ob in range(2):
      pltpu.make_async_copy(
          obufs[ob], out_hbm.at[0, :, pl.ds(bbase, BB)], sem_o[ob]).wait()

  return emb


def kernel(inputs, table):
  NB, T = inputs.shape
  V, d = table.shape
  assert d == D and NB == NW * BB and T % 2 == 0
  idxT = inputs.T
  tabP = jnp.pad(table, ((0, 0), (0, D)))
  out = _make_kernel(T, NB)(idxT, tabP)
  return jnp.transpose(out, (2, 0, 1))
